# baseline (speedup 1.0000x reference)
import functools

import jax
import jax.numpy as jnp
from jax import lax
from jax.experimental import pallas as pl
from jax.experimental.pallas import tpu as pltpu
from jax.experimental.pallas import tpu_sc as plsc

N_GRID = 128
SIDE = 1.5
NPL = N_GRID + 1
NPL2 = NPL * NPL
W_FEAT = 16
B = 524288
C = 256
L = 16
OUT_W = 1 + 3 + W_FEAT

_OFF = [(di, dj, dk) for di in (0, 1) for dj in (0, 1) for dk in (0, 1)]


def _build():
  info = plsc.get_sparse_core_info()
  NC, NS = info.num_cores, info.num_subcores
  NW = NC * NS
  PW = B // NW
  NCHUNK = PW // C
  NIDX = 8 * C

  mesh = plsc.VectorSubcoreMesh(core_axis_name="c", subcore_axis_name="s")

  buf_types = [
      pltpu.VMEM((3 * C,), jnp.float32),
      pltpu.VMEM((NIDX,), jnp.int32),
      pltpu.VMEM((NIDX,), jnp.float32),
      pltpu.VMEM((NIDX, W_FEAT), jnp.float32),
      pltpu.VMEM((NIDX, 1), jnp.float32),
      pltpu.VMEM((C * OUT_W,), jnp.float32),
      pltpu.SemaphoreType.DMA,
      pltpu.SemaphoreType.DMA,
  ]

  @functools.partial(
      pl.kernel,
      mesh=mesh,
      compiler_params=pltpu.CompilerParams(
          needs_layout_passes=False, use_tc_tiling_on_sc=False),
      out_type=jax.ShapeDtypeStruct((B * OUT_W,), jnp.float32),
      scratch_types=buf_types + buf_types,
  )
  def grid_embed(x_hbm, y_hbm, z_hbm, val_hbm, feat_hbm, out_hbm, *scratch):
    bufs = (scratch[:8], scratch[8:])
    wid = lax.axis_index("s") * NC + lax.axis_index("c")
    iota = lax.iota(jnp.int32, L)

    def phase_a_and_fire(t, buf):
      xyz_v, idx_v, w_v, frows_v, vrows_v, out_v, semf, semv = buf
      base = wid * PW + t * C
      pltpu.sync_copy(x_hbm.at[pl.ds(base, C)], xyz_v.at[pl.ds(0, C)])
      pltpu.sync_copy(y_hbm.at[pl.ds(base, C)], xyz_v.at[pl.ds(C, C)])
      pltpu.sync_copy(z_hbm.at[pl.ds(base, C)], xyz_v.at[pl.ds(2 * C, C)])

      for i in range(C // L):
        rows = i * L + iota
        orow = rows * OUT_W
        ix, fr = [], []
        vmask = None
        for d in range(3):
          xd = xyz_v[pl.ds(d * C + i * L, L)]
          ok = (xd >= -0.75) & (xd <= 0.75)
          vmask = ok if vmask is None else (vmask & ok)
          u = (xd + 0.75) / SIDE * float(N_GRID)
          u = jnp.minimum(jnp.maximum(u, 0.0), float(N_GRID))
          ii = jnp.minimum(u.astype(jnp.int32), N_GRID - 1)
          ix.append(ii)
          fr.append(u - ii.astype(jnp.float32))
          plsc.store_scatter(out_v, [orow + (1 + d)], xd)
        validf = jnp.where(vmask, 1.0, 0.0).astype(jnp.float32)
        base_idx = ix[0] * NPL2 + ix[1] * NPL + ix[2]

        for c, (di, dj, dk) in enumerate(_OFF):
          idx_c = base_idx + (di * NPL2 + dj * NPL + dk)
          wc = ((fr[0] if di else 1.0 - fr[0])
                * (fr[1] if dj else 1.0 - fr[1])
                * (fr[2] if dk else 1.0 - fr[2]) * validf)
          idx_v[pl.ds(c * C + i * L, L)] = idx_c
          w_v[pl.ds(c * C + i * L, L)] = wc

      pltpu.async_copy(feat_hbm.at[idx_v], frows_v, semf)
      pltpu.async_copy(val_hbm.at[idx_v], vrows_v, semv)

    def combine_and_write(t, buf):
      xyz_v, idx_v, w_v, frows_v, vrows_v, out_v, semf, semv = buf
      base = wid * PW + t * C
      pltpu.make_async_copy(feat_hbm.at[idx_v], frows_v, semf).wait()
      pltpu.make_async_copy(val_hbm.at[idx_v], vrows_v, semv).wait()

      def combine_body(i, fcarry):
        p0 = i * L
        rows = p0 + iota
        ws = [w_v[pl.ds(c * C + p0, L)] for c in range(8)]

        vacc = jnp.zeros((L,), jnp.float32)
        zeros_i = jnp.zeros((L,), jnp.int32)
        for c in range(8):
          vacc = vacc + ws[c] * plsc.load_gather(
              vrows_v, [c * C + rows, zeros_i])
        plsc.store_scatter(out_v, [rows * OUT_W], vacc)

        for lane in range(L):
          facc = jnp.zeros((W_FEAT,), jnp.float32)
          for c in range(8):
            facc = facc + ws[c][lane] * frows_v[c * C + p0 + lane, :]
          out_v[pl.ds((p0 + lane) * OUT_W + 4, W_FEAT)] = facc
        return fcarry

      lax.fori_loop(0, C // L, combine_body, 0)
      pltpu.sync_copy(out_v, out_hbm.at[pl.ds(base * OUT_W, C * OUT_W)])

    phase_a_and_fire(0, bufs[0])
    phase_a_and_fire(1, bufs[1])

    def pair_body(p, carry):
      t = 2 * p
      combine_and_write(t, bufs[0])

      @pl.when(t + 2 < NCHUNK)
      def _():
        phase_a_and_fire(t + 2, bufs[0])

      combine_and_write(t + 1, bufs[1])

      @pl.when(t + 3 < NCHUNK)
      def _():
        phase_a_and_fire(t + 3, bufs[1])

      return carry

    lax.fori_loop(0, NCHUNK // 2, pair_body, 0)

  return grid_embed


_GRID_EMBED = None


def kernel(xyz, grid_value_param, grid_feat_param):
  global _GRID_EMBED
  if _GRID_EMBED is None:
    _GRID_EMBED = _build()
  x = xyz[:, 0]
  y = xyz[:, 1]
  z = xyz[:, 2]
  out = _GRID_EMBED(x, y, z, grid_value_param, grid_feat_param)
  return jnp.reshape(out, (B, OUT_W))


# speedup vs baseline: 2.9162x; 2.9162x over previous
import functools

import jax
import jax.numpy as jnp
from jax import lax
from jax.experimental import pallas as pl
from jax.experimental.pallas import tpu as pltpu
from jax.experimental.pallas import tpu_sc as plsc

N_GRID = 128
SIDE = 1.5
NPL = N_GRID + 1
NPL2 = NPL * NPL
W_FEAT = 16
B = 524288
C = 256
L = 16
OUT_W = 1 + 3 + W_FEAT

_OFF = [(di, dj, dk) for di in (0, 1) for dj in (0, 1) for dk in (0, 1)]


def _build():
  info = plsc.get_sparse_core_info()
  NC, NS = info.num_cores, info.num_subcores
  NW = NC * NS
  PW = B // NW
  NCHUNK = PW // C
  NIDX = 8 * C

  mesh = plsc.VectorSubcoreMesh(core_axis_name="c", subcore_axis_name="s")

  buf_types = [
      pltpu.VMEM((3 * C,), jnp.float32),
      pltpu.VMEM((NIDX,), jnp.int32),
      pltpu.VMEM((NIDX,), jnp.float32),
      pltpu.VMEM((NIDX, W_FEAT), jnp.float32),
      pltpu.VMEM((NIDX,), jnp.float32),
      pltpu.VMEM((C * OUT_W,), jnp.float32),
      pltpu.SemaphoreType.DMA,
      pltpu.SemaphoreType.DMA,
  ]

  @functools.partial(
      pl.kernel,
      mesh=mesh,
      compiler_params=pltpu.CompilerParams(
          needs_layout_passes=False, use_tc_tiling_on_sc=False),
      out_type=jax.ShapeDtypeStruct((B * OUT_W,), jnp.float32),
      scratch_types=buf_types + buf_types,
  )
  def grid_embed(x_hbm, y_hbm, z_hbm, val_hbm, feat_hbm, out_hbm, *scratch):
    bufs = (scratch[:8], scratch[8:])
    wid = lax.axis_index("s") * NC + lax.axis_index("c")
    iota = lax.iota(jnp.int32, L)

    def phase_a_and_fire(t, buf):
      xyz_v, idx_v, w_v, frows_v, vrows_v, out_v, semf, semv = buf
      base = wid * PW + t * C
      pltpu.sync_copy(x_hbm.at[pl.ds(base, C)], xyz_v.at[pl.ds(0, C)])
      pltpu.sync_copy(y_hbm.at[pl.ds(base, C)], xyz_v.at[pl.ds(C, C)])
      pltpu.sync_copy(z_hbm.at[pl.ds(base, C)], xyz_v.at[pl.ds(2 * C, C)])

      for i in range(C // L):
        rows = i * L + iota
        orow = rows * OUT_W
        ix, fr = [], []
        vmask = None
        for d in range(3):
          xd = xyz_v[pl.ds(d * C + i * L, L)]
          ok = (xd >= -0.75) & (xd <= 0.75)
          vmask = ok if vmask is None else (vmask & ok)
          u = (xd + 0.75) / SIDE * float(N_GRID)
          u = jnp.minimum(jnp.maximum(u, 0.0), float(N_GRID))
          ii = jnp.minimum(u.astype(jnp.int32), N_GRID - 1)
          ix.append(ii)
          fr.append(u - ii.astype(jnp.float32))
          plsc.store_scatter(out_v, [orow + (1 + d)], xd)
        validf = jnp.where(vmask, 1.0, 0.0).astype(jnp.float32)
        base_idx = ix[0] * NPL2 + ix[1] * NPL + ix[2]

        for c, (di, dj, dk) in enumerate(_OFF):
          idx_c = base_idx + (di * NPL2 + dj * NPL + dk)
          wc = ((fr[0] if di else 1.0 - fr[0])
                * (fr[1] if dj else 1.0 - fr[1])
                * (fr[2] if dk else 1.0 - fr[2]) * validf)
          idx_v[pl.ds(c * C + i * L, L)] = idx_c
          w_v[pl.ds(c * C + i * L, L)] = wc

      pltpu.async_copy(feat_hbm.at[idx_v], frows_v, semf)
      pltpu.async_copy(val_hbm.at[idx_v], vrows_v, semv)

    def combine_and_write(t, buf):
      xyz_v, idx_v, w_v, frows_v, vrows_v, out_v, semf, semv = buf
      base = wid * PW + t * C
      pltpu.make_async_copy(feat_hbm.at[idx_v], frows_v, semf).wait()
      pltpu.make_async_copy(val_hbm.at[idx_v], vrows_v, semv).wait()

      def combine_body(i, fcarry):
        p0 = i * L
        rows = p0 + iota
        ws = [w_v[pl.ds(c * C + p0, L)] for c in range(8)]

        vacc = jnp.zeros((L,), jnp.float32)
        for c in range(8):
          vacc = vacc + ws[c] * vrows_v[pl.ds(c * C + p0, L)]
        plsc.store_scatter(out_v, [rows * OUT_W], vacc)

        for lane in range(L):
          facc = jnp.zeros((W_FEAT,), jnp.float32)
          for c in range(8):
            facc = facc + ws[c][lane] * frows_v[c * C + p0 + lane, :]
          out_v[pl.ds((p0 + lane) * OUT_W + 4, W_FEAT)] = facc
        return fcarry

      lax.fori_loop(0, C // L, combine_body, 0)
      pltpu.sync_copy(out_v, out_hbm.at[pl.ds(base * OUT_W, C * OUT_W)])

    phase_a_and_fire(0, bufs[0])
    phase_a_and_fire(1, bufs[1])

    def pair_body(p, carry):
      t = 2 * p
      combine_and_write(t, bufs[0])

      @pl.when(t + 2 < NCHUNK)
      def _():
        phase_a_and_fire(t + 2, bufs[0])

      combine_and_write(t + 1, bufs[1])

      @pl.when(t + 3 < NCHUNK)
      def _():
        phase_a_and_fire(t + 3, bufs[1])

      return carry

    lax.fori_loop(0, NCHUNK // 2, pair_body, 0)

  return grid_embed


_GRID_EMBED = None


def kernel(xyz, grid_value_param, grid_feat_param):
  global _GRID_EMBED
  if _GRID_EMBED is None:
    _GRID_EMBED = _build()
  x = xyz[:, 0]
  y = xyz[:, 1]
  z = xyz[:, 2]
  val = jnp.reshape(grid_value_param, (-1,))
  out = _GRID_EMBED(x, y, z, val, grid_feat_param)
  return jnp.reshape(out, (B, OUT_W))


# speedup vs baseline: 3.0305x; 1.0392x over previous
import functools

import jax
import jax.numpy as jnp
from jax import lax
from jax.experimental import pallas as pl
from jax.experimental.pallas import tpu as pltpu
from jax.experimental.pallas import tpu_sc as plsc

N_GRID = 128
SIDE = 1.5
NPL = N_GRID + 1
NPL2 = NPL * NPL
W_FEAT = 16
B = 524288
C = 256
L = 16
OUT_W = 1 + 3 + W_FEAT

_OFF = [(di, dj, dk) for di in (0, 1) for dj in (0, 1) for dk in (0, 1)]


def _build():
  info = plsc.get_sparse_core_info()
  NC, NS = info.num_cores, info.num_subcores
  NW = NC * NS
  PW = B // NW
  NCHUNK = PW // C
  NIDX = 8 * C

  mesh = plsc.VectorSubcoreMesh(core_axis_name="c", subcore_axis_name="s")

  buf_types = [
      pltpu.VMEM((3 * C,), jnp.float32),
      pltpu.VMEM((NIDX,), jnp.int32),
      pltpu.VMEM((NIDX,), jnp.float32),
      pltpu.VMEM((NIDX, W_FEAT), jnp.float32),
      pltpu.VMEM((NIDX,), jnp.float32),
      pltpu.VMEM((C * OUT_W,), jnp.float32),
      pltpu.SemaphoreType.DMA,
      pltpu.SemaphoreType.DMA,
  ]

  @functools.partial(
      pl.kernel,
      mesh=mesh,
      compiler_params=pltpu.CompilerParams(
          needs_layout_passes=False, use_tc_tiling_on_sc=False),
      out_type=[jax.ShapeDtypeStruct((B * OUT_W // 2,), jnp.float32),
                jax.ShapeDtypeStruct((B * OUT_W // 2,), jnp.float32)],
      scratch_types=buf_types + buf_types,
  )
  def grid_embed(x_hbm, y_hbm, z_hbm, val_hbm, feat_hbm, out0_hbm, out1_hbm,
                 *scratch):
    bufs = (scratch[:8], scratch[8:])
    cid = lax.axis_index("c")
    sid = lax.axis_index("s")
    wid = cid * NS + sid
    iota = lax.iota(jnp.int32, L)

    def phase_a_and_fire(t, buf):
      xyz_v, idx_v, w_v, frows_v, vrows_v, out_v, semf, semv = buf
      base = wid * PW + t * C
      pltpu.sync_copy(x_hbm.at[pl.ds(base, C)], xyz_v.at[pl.ds(0, C)])
      pltpu.sync_copy(y_hbm.at[pl.ds(base, C)], xyz_v.at[pl.ds(C, C)])
      pltpu.sync_copy(z_hbm.at[pl.ds(base, C)], xyz_v.at[pl.ds(2 * C, C)])

      for i in range(C // L):
        rows = i * L + iota
        orow = rows * OUT_W
        ix, fr = [], []
        vmask = None
        for d in range(3):
          xd = xyz_v[pl.ds(d * C + i * L, L)]
          ok = (xd >= -0.75) & (xd <= 0.75)
          vmask = ok if vmask is None else (vmask & ok)
          u = (xd + 0.75) / SIDE * float(N_GRID)
          u = jnp.minimum(jnp.maximum(u, 0.0), float(N_GRID))
          ii = jnp.minimum(u.astype(jnp.int32), N_GRID - 1)
          ix.append(ii)
          fr.append(u - ii.astype(jnp.float32))
          plsc.store_scatter(out_v, [orow + (1 + d)], xd)
        validf = jnp.where(vmask, 1.0, 0.0).astype(jnp.float32)
        base_idx = ix[0] * NPL2 + ix[1] * NPL + ix[2]

        for c, (di, dj, dk) in enumerate(_OFF):
          idx_c = base_idx + (di * NPL2 + dj * NPL + dk)
          wc = ((fr[0] if di else 1.0 - fr[0])
                * (fr[1] if dj else 1.0 - fr[1])
                * (fr[2] if dk else 1.0 - fr[2]) * validf)
          idx_v[pl.ds(c * C + i * L, L)] = idx_c
          w_v[pl.ds(c * C + i * L, L)] = wc

      pltpu.async_copy(feat_hbm.at[idx_v], frows_v, semf)
      pltpu.async_copy(val_hbm.at[idx_v], vrows_v, semv)

    def combine_and_write(t, buf):
      xyz_v, idx_v, w_v, frows_v, vrows_v, out_v, semf, semv = buf
      base = wid * PW + t * C
      pltpu.make_async_copy(feat_hbm.at[idx_v], frows_v, semf).wait()
      pltpu.make_async_copy(val_hbm.at[idx_v], vrows_v, semv).wait()

      def combine_body(i, fcarry):
        p0 = i * L
        rows = p0 + iota
        ws = [w_v[pl.ds(c * C + p0, L)] for c in range(8)]

        vacc = jnp.zeros((L,), jnp.float32)
        for c in range(8):
          vacc = vacc + ws[c] * vrows_v[pl.ds(c * C + p0, L)]
        plsc.store_scatter(out_v, [rows * OUT_W], vacc)

        for lane in range(L):
          facc = jnp.zeros((W_FEAT,), jnp.float32)
          for c in range(8):
            facc = facc + ws[c][lane] * frows_v[c * C + p0 + lane, :]
          out_v[pl.ds((p0 + lane) * OUT_W + 4, W_FEAT)] = facc
        return fcarry

      lax.fori_loop(0, C // L, combine_body, 0)
      local = (sid * PW + t * C) * OUT_W

      @pl.when(cid == 0)
      def _():
        pltpu.sync_copy(out_v, out0_hbm.at[pl.ds(local, C * OUT_W)])

      @pl.when(cid == 1)
      def _():
        pltpu.sync_copy(out_v, out1_hbm.at[pl.ds(local, C * OUT_W)])

    phase_a_and_fire(0, bufs[0])
    phase_a_and_fire(1, bufs[1])

    def pair_body(p, carry):
      t = 2 * p
      combine_and_write(t, bufs[0])

      @pl.when(t + 2 < NCHUNK)
      def _():
        phase_a_and_fire(t + 2, bufs[0])

      combine_and_write(t + 1, bufs[1])

      @pl.when(t + 3 < NCHUNK)
      def _():
        phase_a_and_fire(t + 3, bufs[1])

      return carry

    lax.fori_loop(0, NCHUNK // 2, pair_body, 0)

  return grid_embed


_GRID_EMBED = None


def kernel(xyz, grid_value_param, grid_feat_param):
  global _GRID_EMBED
  if _GRID_EMBED is None:
    _GRID_EMBED = _build()
  x = xyz[:, 0]
  y = xyz[:, 1]
  z = xyz[:, 2]
  val = jnp.reshape(grid_value_param, (-1,))
  out0, out1 = _GRID_EMBED(x, y, z, val, grid_feat_param)
  return jnp.reshape(jnp.concatenate([out0, out1]), (B, OUT_W))
